# baseline (speedup 1.0000x reference)
import jax
import jax.numpy as jnp
from jax import lax
from jax.experimental import pallas as pl
from jax.experimental.pallas import tpu as pltpu
from jax.experimental.pallas import tpu_sc as plsc

N = 100000
NUM_TYPES = 119
C0 = 256
C1 = 128
M1 = 5
LANES = 16

B = 80
NB = N // B
NW = 32
BLK_PER, BLK_REM = NB // NW, NB % NW
IDX_MAIN = BLK_PER * B
IDX_MAX = (BLK_PER + 1) * B

TB = 1000
TNB = N // TB
RING = 4


def _sc_body(f0_hbm, types_hbm, emb0_hbm, out0_hbm,
             idx_all,
             e0a, e0b, e0c, f0a, f0b, f0c,
             sia, sib, sic, soa, sob, soc):
    info = plsc.get_sparse_core_info()
    nc = info.num_cores
    wid = lax.axis_index("s") * nc + lax.axis_index("c")

    e0 = (e0a, e0b, e0c)
    f0 = (f0a, f0b, f0c)
    sin = (sia, sib, sic)
    sout = (soa, sob, soc)

    count = BLK_PER + jnp.where(wid < BLK_REM, 1, 0)
    start_blk = BLK_PER * wid + jnp.minimum(wid, BLK_REM)
    start_atom = start_blk * B

    pltpu.sync_copy(types_hbm.at[pl.ds(start_atom, IDX_MAIN)],
                    idx_all.at[pl.ds(0, IDX_MAIN)])

    @pl.when(count == BLK_PER + 1)
    def _():
        pltpu.sync_copy(types_hbm.at[pl.ds(start_atom + IDX_MAIN, B)],
                        idx_all.at[pl.ds(IDX_MAIN, B)])

    def in_descs(j, r):
        base = (start_blk + j) * B
        idx_ref = idx_all.at[pl.ds(j * B, B)]
        return (
            pltpu.make_async_copy(emb0_hbm.at[idx_ref], e0[r], sin[r]),
            pltpu.make_async_copy(f0_hbm.at[pl.ds(base, B)], f0[r], sin[r]),
        )

    def out_descs(j, r):
        base = (start_blk + j) * B
        return (
            pltpu.make_async_copy(f0[r], out0_hbm.at[pl.ds(base, B)], sout[r]),
        )

    def prefetch(j, r):
        for d in in_descs(j, r):
            d.start()

    def wait_in(j, r):
        for d in in_descs(j, r):
            d.wait()

    def issue_out(j, r):
        for d in out_descs(j, r):
            d.start()

    def wait_out(j, r):
        for d in out_descs(j, r):
            d.wait()

    def compute(r):
        f0r, e0r = f0[r], e0[r]

        def row_body(row, rc):
            for c in range(C0 // LANES):
                sl = pl.ds(c * LANES, LANES)
                f0r[row, sl] = f0r[row, sl] * e0r[row, sl]
            return rc

        lax.fori_loop(0, B, row_body, 0)

    prefetch(0, 0)

    def outer(g, carry):
        for b in range(3):
            j = g * 3 + b
            rn = (b + 1) % 3

            @pl.when(j + 1 < count)
            def _():
                @pl.when(j >= 2)
                def _():
                    wait_out(j - 2, rn)
                prefetch(j + 1, rn)

            @pl.when(j < count)
            def _():
                wait_in(j, b)
                compute(b)
                issue_out(j, b)
        return carry

    lax.fori_loop(0, (count + 2) // 3, outer, 0)

    for r in range(3):
        last = jnp.where((count - 1) % 3 == r, count - 1, count - 2)
        pending = jnp.logical_or((count - 1) % 3 == r, (count - 2) % 3 == r)

        @pl.when(pending)
        def _():
            wait_out(last, r)


def _tc_body(types_ref, emb1_ref, f1_hbm, out1_hbm, fbuf, obuf, sin, sout):
    def in_copy(j, r):
        return pltpu.make_async_copy(
            f1_hbm.at[:, pl.ds(j * TB, TB), :], fbuf.at[r], sin.at[r])

    def out_copy(j, r):
        return pltpu.make_async_copy(
            obuf.at[r], out1_hbm.at[:, pl.ds(j * TB, TB), :], sout.at[r])

    def compute(j, r):
        t = types_ref[j, :]
        onehot = (t[:, None] == lax.broadcasted_iota(
            jnp.int32, (TB, NUM_TYPES), 1)).astype(jnp.float32)
        e1r = jnp.dot(onehot, emb1_ref[...],
                      preferred_element_type=jnp.float32,
                      precision=lax.Precision.HIGHEST)
        for m in range(M1):
            obuf[r, m, :, :] = fbuf[r, m, :, :] * e1r

    for b in range(RING - 1):
        in_copy(b, b).start()

    def outer(g, carry):
        for b in range(RING):
            j = g * RING + b

            @pl.when(j + RING - 1 < TNB)
            def _():
                in_copy(j + RING - 1, (b + RING - 1) % RING).start()

            in_copy(j, b).wait()

            @pl.when(j >= RING)
            def _():
                out_copy(j - RING, b).wait()

            compute(j, b)
            out_copy(j, b).start()
        return carry

    lax.fori_loop(0, TNB // RING, outer, 0, unroll=False)

    for b in range(RING):
        out_copy(TNB - RING + b, b).wait()


def kernel(features_0, features_1, atomic_types, emb_0, emb_1):
    types32 = atomic_types.astype(jnp.int32)

    sc = pl.kernel(
        _sc_body,
        mesh=plsc.VectorSubcoreMesh(core_axis_name="c", subcore_axis_name="s"),
        out_type=[jax.ShapeDtypeStruct((N, C0), jnp.float32)],
        scratch_types=(
            [pltpu.VMEM((IDX_MAX,), jnp.int32)]
            + [pltpu.VMEM((B, C0), jnp.float32) for _ in range(3)]
            + [pltpu.VMEM((B, C0), jnp.float32) for _ in range(3)]
            + [pltpu.SemaphoreType.DMA for _ in range(6)]
        ),
    )
    (out0,) = sc(features_0, types32, emb_0)

    types2d = types32.reshape(TNB, TB)
    f1t = jnp.transpose(features_1, (1, 0, 2))
    out1t = pl.pallas_call(
        _tc_body,
        in_specs=[
            pl.BlockSpec(memory_space=pltpu.VMEM),
            pl.BlockSpec(memory_space=pltpu.VMEM),
            pl.BlockSpec(memory_space=pl.ANY),
        ],
        out_specs=pl.BlockSpec(memory_space=pl.ANY),
        out_shape=jax.ShapeDtypeStruct((M1, N, C1), jnp.float32),
        scratch_shapes=[
            pltpu.VMEM((RING, M1, TB, C1), jnp.float32),
            pltpu.VMEM((RING, M1, TB, C1), jnp.float32),
            pltpu.SemaphoreType.DMA((RING,)),
            pltpu.SemaphoreType.DMA((RING,)),
        ],
    )(types2d, emb_1, f1t)
    out1 = jnp.transpose(out1t, (1, 0, 2))

    return (out0, out1)


# speedup vs baseline: 1.0089x; 1.0089x over previous
import jax
import jax.numpy as jnp
from jax import lax
from jax.experimental import pallas as pl
from jax.experimental.pallas import tpu as pltpu
from jax.experimental.pallas import tpu_sc as plsc

N = 100000
NUM_TYPES = 119
C0 = 256
C1 = 128
M1 = 5
LANES = 16

B = 80
NB = N // B
NW = 32
BLK_PER, BLK_REM = NB // NW, NB % NW
IDX_MAIN = BLK_PER * B
IDX_MAX = (BLK_PER + 1) * B

TB = 1000
TNB = N // TB
RING = 4
CB = 2000
CNB = N // CB


def _sc_body(f0_hbm, types_hbm, emb0_hbm, out0_hbm,
             idx_all,
             e0a, e0b, e0c, f0a, f0b, f0c,
             sia, sib, sic, soa, sob, soc):
    info = plsc.get_sparse_core_info()
    nc = info.num_cores
    wid = lax.axis_index("s") * nc + lax.axis_index("c")

    e0 = (e0a, e0b, e0c)
    f0 = (f0a, f0b, f0c)
    sin = (sia, sib, sic)
    sout = (soa, sob, soc)

    count = BLK_PER + jnp.where(wid < BLK_REM, 1, 0)
    start_blk = BLK_PER * wid + jnp.minimum(wid, BLK_REM)
    start_atom = start_blk * B

    pltpu.sync_copy(types_hbm.at[pl.ds(start_atom, IDX_MAIN)],
                    idx_all.at[pl.ds(0, IDX_MAIN)])

    @pl.when(count == BLK_PER + 1)
    def _():
        pltpu.sync_copy(types_hbm.at[pl.ds(start_atom + IDX_MAIN, B)],
                        idx_all.at[pl.ds(IDX_MAIN, B)])

    def in_descs(j, r):
        base = (start_blk + j) * B
        idx_ref = idx_all.at[pl.ds(j * B, B)]
        return (
            pltpu.make_async_copy(emb0_hbm.at[idx_ref], e0[r], sin[r]),
            pltpu.make_async_copy(f0_hbm.at[pl.ds(base, B)], f0[r], sin[r]),
        )

    def out_descs(j, r):
        base = (start_blk + j) * B
        return (
            pltpu.make_async_copy(f0[r], out0_hbm.at[pl.ds(base, B)], sout[r]),
        )

    def prefetch(j, r):
        for d in in_descs(j, r):
            d.start()

    def wait_in(j, r):
        for d in in_descs(j, r):
            d.wait()

    def issue_out(j, r):
        for d in out_descs(j, r):
            d.start()

    def wait_out(j, r):
        for d in out_descs(j, r):
            d.wait()

    def compute(r):
        f0r, e0r = f0[r], e0[r]

        def row_body(row, rc):
            for c in range(C0 // LANES):
                sl = pl.ds(c * LANES, LANES)
                f0r[row, sl] = f0r[row, sl] * e0r[row, sl]
            return rc

        lax.fori_loop(0, B, row_body, 0)

    prefetch(0, 0)

    def outer(g, carry):
        for b in range(3):
            j = g * 3 + b
            rn = (b + 1) % 3

            @pl.when(j + 1 < count)
            def _():
                @pl.when(j >= 2)
                def _():
                    wait_out(j - 2, rn)
                prefetch(j + 1, rn)

            @pl.when(j < count)
            def _():
                wait_in(j, b)
                compute(b)
                issue_out(j, b)
        return carry

    lax.fori_loop(0, (count + 2) // 3, outer, 0)

    for r in range(3):
        last = jnp.where((count - 1) % 3 == r, count - 1, count - 2)
        pending = jnp.logical_or((count - 1) % 3 == r, (count - 2) % 3 == r)

        @pl.when(pending)
        def _():
            wait_out(last, r)


def _tc_body_classic(types_ref, emb1_ref, f1_ref, out1_ref):
    t = types_ref[0, 0, :]
    onehot = (t[:, None] == lax.broadcasted_iota(
        jnp.int32, (CB, NUM_TYPES), 1)).astype(jnp.float32)
    e1r = jnp.dot(onehot, emb1_ref[...],
                  preferred_element_type=jnp.float32,
                  precision=lax.Precision.HIGHEST)
    for m in range(M1):
        out1_ref[m, :, :] = f1_ref[m, :, :] * e1r


def _tc_body(types_ref, emb1_ref, f1_hbm, out1_hbm, fbuf, obuf, sin, sout):
    def in_copy(j, r):
        return pltpu.make_async_copy(
            f1_hbm.at[:, pl.ds(j * TB, TB), :], fbuf.at[r], sin.at[r])

    def out_copy(j, r):
        return pltpu.make_async_copy(
            obuf.at[r], out1_hbm.at[:, pl.ds(j * TB, TB), :], sout.at[r])

    def compute(j, r):
        t = types_ref[j, :]
        onehot = (t[:, None] == lax.broadcasted_iota(
            jnp.int32, (TB, NUM_TYPES), 1)).astype(jnp.float32)
        e1r = jnp.dot(onehot, emb1_ref[...],
                      preferred_element_type=jnp.float32,
                      precision=lax.Precision.HIGHEST)
        for m in range(M1):
            obuf[r, m, :, :] = fbuf[r, m, :, :] * e1r

    for b in range(RING - 1):
        in_copy(b, b).start()

    def outer(g, carry):
        for b in range(RING):
            j = g * RING + b

            @pl.when(j + RING - 1 < TNB)
            def _():
                in_copy(j + RING - 1, (b + RING - 1) % RING).start()

            in_copy(j, b).wait()

            @pl.when(j >= RING)
            def _():
                out_copy(j - RING, b).wait()

            compute(j, b)
            out_copy(j, b).start()
        return carry

    lax.fori_loop(0, TNB // RING, outer, 0, unroll=False)

    for b in range(RING):
        out_copy(TNB - RING + b, b).wait()


def kernel(features_0, features_1, atomic_types, emb_0, emb_1):
    types32 = atomic_types.astype(jnp.int32)

    sc = pl.kernel(
        _sc_body,
        mesh=plsc.VectorSubcoreMesh(core_axis_name="c", subcore_axis_name="s"),
        out_type=[jax.ShapeDtypeStruct((N, C0), jnp.float32)],
        scratch_types=(
            [pltpu.VMEM((IDX_MAX,), jnp.int32)]
            + [pltpu.VMEM((B, C0), jnp.float32) for _ in range(3)]
            + [pltpu.VMEM((B, C0), jnp.float32) for _ in range(3)]
            + [pltpu.SemaphoreType.DMA for _ in range(6)]
        ),
    )
    (out0,) = sc(features_0, types32, emb_0)

    types3d = types32.reshape(CNB, 1, CB)
    f1t = jnp.transpose(features_1, (1, 0, 2))
    out1t = pl.pallas_call(
        _tc_body_classic,
        grid=(CNB,),
        in_specs=[
            pl.BlockSpec((1, 1, CB), lambda i: (i, 0, 0)),
            pl.BlockSpec((NUM_TYPES, C1), lambda i: (0, 0)),
            pl.BlockSpec((M1, CB, C1), lambda i: (0, i, 0)),
        ],
        out_specs=pl.BlockSpec((M1, CB, C1), lambda i: (0, i, 0)),
        out_shape=jax.ShapeDtypeStruct((M1, N, C1), jnp.float32),
    )(types3d, emb_1, f1t)
    out1 = jnp.transpose(out1t, (1, 0, 2))

    return (out0, out1)
